# baseline (speedup 1.0000x reference)
import jax
import jax.numpy as jnp
from jax import lax
from jax.experimental import pallas as pl
from jax.experimental.pallas import tpu as pltpu
from jax.experimental.pallas import tpu_sc as plsc

_B, _C, _H, _W = 4, 192, 224, 224
_ROWS = _B * _C
_ROWLEN = _H * _W
_SUB = 8
_SUBLEN = _ROWLEN // _SUB
_NWORKERS = 32
_RPW = _ROWS // _NWORKERS
_CPB = 24


def _sc_body(x_hbm, perm_hbm, o_hbm, perm_v, idxA, idxB, buf0, buf1, gsem, ssem):
    cid = lax.axis_index("c")
    sid = lax.axis_index("s")
    wid = cid * 16 + sid
    b = wid // 8
    c0 = _CPB * (wid % 8)
    base_out = _RPW * wid

    pltpu.sync_copy(perm_hbm.at[pl.ds(c0, _CPB)], perm_v)
    lo = perm_v[pl.ds(0, 16)] + b * _C
    hi = perm_v[pl.ds(8, 16)] + b * _C
    lane = lax.iota(jnp.int32, 16)
    half = lane < 8

    def src_plane(j):
        return lo[j] if j < 16 else hi[j - 8]

    idx = (idxA, idxB)
    buf = (buf0, buf1)

    def write_pair_idx(p):
        a = src_plane(2 * p) * _SUB
        bb = src_plane(2 * p + 1) * _SUB - 8
        idx[p % 2][...] = jnp.where(half, a, bb) + lane

    def gather(j):
        s = j % 2
        ilist = idx[(j // 2) % 2].at[pl.ds((j % 2) * 8, 8)]
        pltpu.make_async_copy(x_hbm.at[ilist], buf[s], gsem.at[s]).start()

    def gather_wait(j):
        s = j % 2
        ilist = idx[(j // 2) % 2].at[pl.ds((j % 2) * 8, 8)]
        pltpu.make_async_copy(x_hbm.at[ilist], buf[s], gsem.at[s]).wait()

    def scatter(j, start):
        s = j % 2
        cp = pltpu.make_async_copy(
            buf[s], o_hbm.at[pl.ds((base_out + j) * _SUB, _SUB)], ssem.at[s]
        )
        cp.start() if start else cp.wait()

    for t in range(_RPW + 1):
        if t < _RPW:
            if t >= 2:
                scatter(t - 2, start=False)
            if t % 2 == 0:
                write_pair_idx(t // 2)
            gather(t)
        d = t - 1
        if d >= 0:
            gather_wait(d)
            scatter(d, start=True)
    scatter(_RPW - 2, start=False)
    scatter(_RPW - 1, start=False)


def kernel(x, ldj, permutation):
    B, C, H, W = x.shape
    x8 = x.reshape(B * C * _SUB, _SUBLEN)
    k = pl.kernel(
        _sc_body,
        out_type=jax.ShapeDtypeStruct((B * C * _SUB, _SUBLEN), x.dtype),
        mesh=plsc.VectorSubcoreMesh(core_axis_name="c", subcore_axis_name="s"),
        scratch_types=[
            pltpu.VMEM((_CPB,), jnp.int32),
            pltpu.VMEM((16,), jnp.int32),
            pltpu.VMEM((16,), jnp.int32),
            pltpu.VMEM((_SUB, _SUBLEN), jnp.float32),
            pltpu.VMEM((_SUB, _SUBLEN), jnp.float32),
            pltpu.SemaphoreType.DMA((2,)),
            pltpu.SemaphoreType.DMA((2,)),
        ],
    )
    out = k(x8, permutation)
    return out.reshape(B, C, H, W), ldj


# speedup vs baseline: 3.9372x; 3.9372x over previous
import jax
import jax.numpy as jnp
from jax.experimental import pallas as pl
from jax.experimental.pallas import tpu as pltpu


def _copy2_body(perm_ref, x0_ref, x1_ref, o_ref):
    o_ref[:, 0:1] = x0_ref[...]
    o_ref[:, 1:2] = x1_ref[...]


def kernel(x, ldj, permutation):
    B, C, H, W = x.shape
    out = pl.pallas_call(
        _copy2_body,
        grid_spec=pltpu.PrefetchScalarGridSpec(
            num_scalar_prefetch=1,
            grid=(C // 2,),
            in_specs=[
                pl.BlockSpec((B, 1, H, W), lambda i, perm: (0, perm[2 * i], 0, 0)),
                pl.BlockSpec((B, 1, H, W), lambda i, perm: (0, perm[2 * i + 1], 0, 0)),
            ],
            out_specs=pl.BlockSpec((B, 2, H, W), lambda i, perm: (0, i, 0, 0)),
        ),
        out_shape=jax.ShapeDtypeStruct((B, C, H, W), x.dtype),
        compiler_params=pltpu.CompilerParams(
            dimension_semantics=("arbitrary",),
        ),
    )(permutation, x, x)
    return out, ldj


# speedup vs baseline: 4.3760x; 1.1114x over previous
import jax
import jax.numpy as jnp
from jax.experimental import pallas as pl
from jax.experimental.pallas import tpu as pltpu

_NSTREAM = 4


def _copy4_body(perm_ref, *refs):
    o_ref = refs[-1]
    for s in range(_NSTREAM):
        o_ref[:, s : s + 1] = refs[s][...]


def _in_spec(s):
    return pl.BlockSpec(
        (4, 1, 224, 224), lambda i, perm: (0, perm[_NSTREAM * i + s], 0, 0)
    )


def kernel(x, ldj, permutation):
    B, C, H, W = x.shape
    out = pl.pallas_call(
        _copy4_body,
        grid_spec=pltpu.PrefetchScalarGridSpec(
            num_scalar_prefetch=1,
            grid=(C // _NSTREAM,),
            in_specs=[_in_spec(s) for s in range(_NSTREAM)],
            out_specs=pl.BlockSpec(
                (B, _NSTREAM, H, W), lambda i, perm: (0, i, 0, 0)
            ),
        ),
        out_shape=jax.ShapeDtypeStruct((B, C, H, W), x.dtype),
        compiler_params=pltpu.CompilerParams(
            dimension_semantics=("arbitrary",),
        ),
    )(permutation, *([x] * _NSTREAM))
    return out, ldj


# speedup vs baseline: 4.5295x; 1.0351x over previous
import jax
import jax.numpy as jnp
from jax.experimental import pallas as pl
from jax.experimental.pallas import tpu as pltpu

_NSTREAM = 8


def _copy4_body(perm_ref, *refs):
    o_ref = refs[-1]
    for s in range(_NSTREAM):
        o_ref[:, s : s + 1] = refs[s][...]


def _in_spec(s):
    return pl.BlockSpec(
        (4, 1, 224, 224), lambda i, perm: (0, perm[_NSTREAM * i + s], 0, 0)
    )


def kernel(x, ldj, permutation):
    B, C, H, W = x.shape
    out = pl.pallas_call(
        _copy4_body,
        grid_spec=pltpu.PrefetchScalarGridSpec(
            num_scalar_prefetch=1,
            grid=(C // _NSTREAM,),
            in_specs=[_in_spec(s) for s in range(_NSTREAM)],
            out_specs=pl.BlockSpec(
                (B, _NSTREAM, H, W), lambda i, perm: (0, i, 0, 0)
            ),
        ),
        out_shape=jax.ShapeDtypeStruct((B, C, H, W), x.dtype),
        compiler_params=pltpu.CompilerParams(
            dimension_semantics=("arbitrary",),
        ),
    )(permutation, *([x] * _NSTREAM))
    return out, ldj
